# baseline (speedup 1.0000x reference)
import functools
import math

import jax
import jax.numpy as jnp
from jax import lax
from jax.experimental import pallas as pl
from jax.experimental.pallas import tpu as pltpu
from jax.experimental.pallas import tpu_sc as plsc

N = 10000
E = 320000
D = 128
DE = 16
HID = 128

NC = 2
NS = 16
NW = NC * NS
C = 80
SLICES = (166400, 153600)
BE = 3200
NACC = 10240
NPT = NACC // NS

_mesh = plsc.VectorSubcoreMesh(core_axis_name="c", subcore_axis_name="s")
_sc_params = pltpu.CompilerParams(needs_layout_passes=False,
                                  use_tc_tiling_on_sc=False)


def _make_gather(ne):
    ept = ne // NW
    nchunk = ept // C

    @functools.partial(
        pl.kernel,
        out_type=(
            jax.ShapeDtypeStruct((ne, D), jnp.bfloat16),
            jax.ShapeDtypeStruct((ne * 4,), jnp.float32),
            jax.ShapeDtypeStruct((ne,), jnp.float32),
        ),
        mesh=_mesh,
        scratch_types=[
            pltpu.VMEM((nchunk, C), jnp.int32),
            pltpu.VMEM((nchunk, C), jnp.int32),
            [pltpu.VMEM((C, D), jnp.bfloat16)] * 2,
            [pltpu.VMEM((C, D), jnp.bfloat16)] * 2,
            [pltpu.VMEM((C, 16), jnp.float32)] * 2,
            [pltpu.VMEM((C, 16), jnp.float32)] * 2,
            [pltpu.VMEM((C, D), jnp.bfloat16)] * 2,
            [pltpu.VMEM((C * 4,), jnp.float32)] * 2,
            [pltpu.VMEM((C,), jnp.float32)] * 2,
            [pltpu.SemaphoreType.DMA] * 2,
            [pltpu.SemaphoreType.DMA] * 2,
        ],
        compiler_params=_sc_params,
    )
    def _gather_k(hA, hB, y16, si3, ri3, g_out, cd_out, rad_out,
                  siv, riv, bA, bB, bYs, bYr, gst, cdb, radv, gsem, wsem):
        cid = lax.axis_index("c")
        sid = lax.axis_index("s")
        wid = sid * NC + cid
        base = wid * ept
        pltpu.sync_copy(si3.at[wid], siv)
        pltpu.sync_copy(ri3.at[wid], riv)

        def issue(k, s):
            pltpu.async_copy(hA.at[siv.at[k]], bA[s], gsem[s])
            pltpu.async_copy(hB.at[riv.at[k]], bB[s], gsem[s])
            pltpu.async_copy(y16.at[siv.at[k]], bYs[s], gsem[s])
            pltpu.async_copy(y16.at[riv.at[k]], bYr[s], gsem[s])

        def step(k, s):
            pltpu.make_async_copy(hA.at[pl.ds(0, C)], bA[s], gsem[s]).wait()
            pltpu.make_async_copy(hB.at[pl.ds(0, C)], bB[s], gsem[s]).wait()
            pltpu.make_async_copy(y16.at[pl.ds(0, C)], bYs[s], gsem[s]).wait()
            pltpu.make_async_copy(y16.at[pl.ds(0, C)], bYr[s], gsem[s]).wait()

            @pl.when(k >= 2)
            def _():
                pltpu.make_async_copy(gst[s], g_out.at[pl.ds(0, C)],
                                      wsem[s]).wait()
                pltpu.make_async_copy(cdb[s], cd_out.at[pl.ds(0, C * 4)],
                                      wsem[s]).wait()
                pltpu.make_async_copy(radv[s], rad_out.at[pl.ds(0, C)],
                                      wsem[s]).wait()

            def cgrp(t, carry2):
                rows = lax.iota(jnp.int32, 16) + t * 16
                rad = jnp.zeros((16,), jnp.float32)
                for comp in range(3):
                    cvec = jnp.full((16,), comp, jnp.int32)
                    ys = plsc.load_gather(bYs[s], [rows, cvec])
                    yr = plsc.load_gather(bYr[s], [rows, cvec])
                    dd = yr - ys
                    plsc.store_scatter(cdb[s], [rows * 4 + comp], dd)
                    rad = rad + dd * dd
                plsc.store_scatter(cdb[s], [rows * 4 + 3], rad)
                radv[s][pl.ds(t * 16, 16)] = rad
                return carry2
            lax.fori_loop(0, C // 16, cgrp, 0)

            def addrow(i, carry2):
                for j in range(D // 32):
                    gst[s][i, pl.ds(j * 32, 32)] = (
                        bA[s][i, pl.ds(j * 32, 32)]
                        + bB[s][i, pl.ds(j * 32, 32)])
                return carry2
            lax.fori_loop(0, C, addrow, 0)

            off = base + k * C
            pltpu.async_copy(gst[s], g_out.at[pl.ds(off, C)], wsem[s])
            pltpu.async_copy(cdb[s], cd_out.at[pl.ds(off * 4, C * 4)],
                             wsem[s])
            pltpu.async_copy(radv[s], rad_out.at[pl.ds(off, C)], wsem[s])

            @pl.when(k + 2 < nchunk)
            def _():
                issue(k + 2, s)

        issue(0, 0)
        issue(1, 1)

        def pair(i, carry):
            step(2 * i, 0)

            @pl.when(2 * i + 1 < nchunk)
            def _():
                step(2 * i + 1, 1)
            return carry
        lax.fori_loop(0, (nchunk + 1) // 2, pair, 0)
        for s in range(2):
            pltpu.make_async_copy(gst[s], g_out.at[pl.ds(0, C)],
                                  wsem[s]).wait()
            pltpu.make_async_copy(cdb[s], cd_out.at[pl.ds(0, C * 4)],
                                  wsem[s]).wait()
            pltpu.make_async_copy(radv[s], rad_out.at[pl.ds(0, C)],
                                  wsem[s]).wait()

    return _gather_k


_gathers = tuple(_make_gather(ne) for ne in SLICES)


def _make_scatter(ne):
    ept = ne // NW
    nchunk = ept // C

    @functools.partial(
        pl.kernel,
        out_type=(
            jax.ShapeDtypeStruct((NC, NACC, HID), jnp.float32),
            jax.ShapeDtypeStruct((NC, NACC, 16), jnp.float32),
        ),
        mesh=_mesh,
        scratch_types=[
            pltpu.VMEM((nchunk, C), jnp.int32),
            [pltpu.VMEM((C, HID), jnp.float32)] * 2,
            [pltpu.VMEM((C, 16), jnp.float32)] * 2,
            [pltpu.VMEM((C,), jnp.float32)] * 2,
            [pltpu.VMEM((C * 4,), jnp.float32)] * 2,
            [pltpu.SemaphoreType.DMA] * 2,
            pltpu.VMEM_SHARED((NACC, HID), jnp.float32),
            pltpu.VMEM_SHARED((NACC, 16), jnp.float32),
        ],
        compiler_params=_sc_params,
    )
    def _scatter_k(ri3, mij, w, cd, accm_out, acct_out,
                   riv, mb, tb, wb, cb, lsem, accm_sh, acct_sh):
        cid = lax.axis_index("c")
        sid = lax.axis_index("s")
        wid = sid * NC + cid
        base = wid * ept
        pltpu.sync_copy(ri3.at[wid], riv)

        def zrow(i, carry):
            for j in range(HID // 16):
                mb[0][i, pl.ds(j * 16, 16)] = jnp.zeros((16,), jnp.float32)
            tb[0][i, pl.ds(0, 16)] = jnp.zeros((16,), jnp.float32)
            tb[1][i, pl.ds(0, 16)] = jnp.zeros((16,), jnp.float32)
            return carry
        lax.fori_loop(0, C, zrow, 0)

        for q in range(NPT // C):
            pltpu.sync_copy(mb[0], accm_sh.at[pl.ds(sid * NPT + q * C, C)])
            pltpu.sync_copy(tb[0], acct_sh.at[pl.ds(sid * NPT + q * C, C)])
        plsc.subcore_barrier()

        def issue(k, s):
            off = base + k * C
            pltpu.async_copy(mij.at[pl.ds(off, C)], mb[s], lsem[s])
            pltpu.async_copy(w.at[pl.ds(off, C)], wb[s], lsem[s])
            pltpu.async_copy(cd.at[pl.ds(off * 4, C * 4)], cb[s], lsem[s])

        def step(k, s):
            pltpu.make_async_copy(mij.at[pl.ds(0, C)], mb[s], lsem[s]).wait()
            pltpu.make_async_copy(w.at[pl.ds(0, C)], wb[s], lsem[s]).wait()
            pltpu.make_async_copy(cd.at[pl.ds(0, C * 4)], cb[s],
                                  lsem[s]).wait()

            def tgrp(t, carry2):
                rows = lax.iota(jnp.int32, 16) + t * 16
                wv = wb[s][pl.ds(t * 16, 16)]
                for comp in range(3):
                    dd = plsc.load_gather(cb[s], [rows * 4 + comp])
                    plsc.store_scatter(
                        tb[s], [rows, jnp.full((16,), comp, jnp.int32)],
                        dd * wv)
                plsc.store_scatter(
                    tb[s], [rows, jnp.full((16,), 3, jnp.int32)],
                    jnp.full((16,), 1.0, jnp.float32))
                return carry2
            lax.fori_loop(0, C // 16, tgrp, 0)

            pltpu.sync_copy(mb[s], accm_sh.at[riv.at[k]], add=True)
            pltpu.sync_copy(tb[s], acct_sh.at[riv.at[k]], add=True)

            @pl.when(k + 2 < nchunk)
            def _():
                issue(k + 2, s)

        issue(0, 0)
        issue(1, 1)

        def pair(i, carry):
            step(2 * i, 0)

            @pl.when(2 * i + 1 < nchunk)
            def _():
                step(2 * i + 1, 1)
            return carry
        lax.fori_loop(0, (nchunk + 1) // 2, pair, 0)
        plsc.subcore_barrier()

        for q in range(NPT // C):
            rows = sid * NPT + q * C
            pltpu.sync_copy(accm_sh.at[pl.ds(rows, C)], mb[0])
            pltpu.sync_copy(mb[0], accm_out.at[cid, pl.ds(rows, C)])
            pltpu.sync_copy(acct_sh.at[pl.ds(rows, C)], tb[0])
            pltpu.sync_copy(tb[0], acct_out.at[cid, pl.ds(rows, C)])

    return _scatter_k


_scatters = tuple(_make_scatter(ne) for ne in SLICES)


def _pre_body(h_ref, wa_ref, wb_ref, ha_ref, hb_ref):
    h = h_ref[...]
    ha_ref[...] = jnp.dot(h, wa_ref[...],
                          preferred_element_type=jnp.float32
                          ).astype(jnp.bfloat16)
    hb_ref[...] = jnp.dot(h, wb_ref[...],
                          preferred_element_type=jnp.float32
                          ).astype(jnp.bfloat16)


def _edge_body(g_ref, eat_ref, rad_ref, wea_ref, be1_ref,
               we2_ref, be2_ref, wc1_ref, bc1_ref, wc2_ref,
               mij_ref, w_ref):
    g = g_ref[...].astype(jnp.float32)
    be = g.shape[0]
    i = pl.program_id(0)
    rad_row = rad_ref[pl.ds(i * be, be)].reshape(1, be)
    ea17 = jnp.concatenate([eat_ref[...], rad_row], axis=0)
    pre1 = (g
            + jax.lax.dot_general(ea17, wea_ref[...],
                                  (((0,), (0,)), ((), ())),
                                  preferred_element_type=jnp.float32)
            + be1_ref[...])
    m1 = jax.nn.silu(pre1)
    mij = jax.nn.silu(
        jnp.dot(m1.astype(jnp.bfloat16), we2_ref[...],
                preferred_element_type=jnp.float32)
        + be2_ref[...])
    cvec = jax.nn.silu(
        jnp.dot(mij.astype(jnp.bfloat16), wc1_ref[...],
                preferred_element_type=jnp.float32)
        + bc1_ref[...])
    w_row = jax.lax.dot_general(wc2_ref[...], cvec,
                                (((1,), (1,)), ((), ())),
                                preferred_element_type=jnp.float32)
    mij_ref[...] = mij
    w_ref[pl.ds(i * be, be)] = w_row.reshape(be)


def _node_body(h_ref, y4_ref, *rest):
    k = 2 * len(SLICES)
    ams = rest[:k]
    ats = rest[k:2 * k]
    wn1t_ref, wn1b_ref, bn1_ref, wn2_ref, bn2_ref, hout_ref, yout_ref = \
        rest[2 * k:]
    h = h_ref[...]
    mi = ams[0][0]
    for r in ams[1:]:
        mi = mi + r[0]
    mi = mi * (1.0 / math.sqrt(648.0))
    u = jax.nn.silu(
        jnp.dot(h, wn1t_ref[...], preferred_element_type=jnp.float32)
        + jnp.dot(mi, wn1b_ref[...], preferred_element_type=jnp.float32)
        + bn1_ref[...])
    hout_ref[...] = (h + jnp.dot(u, wn2_ref[...],
                                 preferred_element_type=jnp.float32)
                     + bn2_ref[...])
    t = ats[0][0]
    for r in ats[1:]:
        t = t + r[0]
    cnt = jnp.maximum(t[:, 3:4], 1.0)
    yout_ref[...] = y4_ref[...] + t[:, :4] / cnt


def _full(shape):
    return pl.BlockSpec(shape, lambda i: (0,) * len(shape))


def _edge_mlp(g, eaT, rad, Wea17, be1, We2, be2, Wc1, bc1, Wc2):
    ne = g.shape[0]
    return pl.pallas_call(
        _edge_body,
        grid=(ne // BE,),
        in_specs=[pl.BlockSpec((BE, HID), lambda i: (i, 0)),
                  pl.BlockSpec((DE, BE), lambda i: (0, i)),
                  pl.BlockSpec((ne,), lambda i: (0,)),
                  _full((DE + 1, HID)), _full((1, HID)),
                  pl.BlockSpec((HID, HID), lambda i: (0, 0)),
                  _full((1, HID)),
                  pl.BlockSpec((HID, HID), lambda i: (0, 0)),
                  _full((1, HID)), _full((1, HID))],
        out_specs=[pl.BlockSpec((BE, HID), lambda i: (i, 0)),
                   pl.BlockSpec((ne,), lambda i: (0,))],
        out_shape=[jax.ShapeDtypeStruct((ne, HID), jnp.float32),
                   jax.ShapeDtypeStruct((ne,), jnp.float32)],
    )(g, eaT, rad, Wea17, be1, We2, be2, Wc1, bc1, Wc2)


def kernel(h, edge_index, y, edge_attr, We1, be1, We2, be2,
           Wc1, bc1, Wc2, Wn1, bn1, Wn2, bn2):
    receivers = edge_index[0].astype(jnp.int32)
    senders = edge_index[1].astype(jnp.int32)
    y4 = jnp.pad(y, ((0, 0), (0, 1)))
    y16 = jnp.pad(y, ((0, 0), (0, 13)))

    WA = We1[:D]
    WB = We1[D:2 * D]
    Wea17 = jnp.concatenate([We1[2 * D + 1:], We1[2 * D:2 * D + 1]], axis=0)

    BN = 2000
    hA, hB = pl.pallas_call(
        _pre_body,
        grid=(N // BN,),
        in_specs=[pl.BlockSpec((BN, D), lambda i: (i, 0)),
                  _full((D, HID)), _full((D, HID))],
        out_specs=[pl.BlockSpec((BN, HID), lambda i: (i, 0)),
                   pl.BlockSpec((BN, HID), lambda i: (i, 0))],
        out_shape=[jax.ShapeDtypeStruct((N, HID), jnp.bfloat16),
                   jax.ShapeDtypeStruct((N, HID), jnp.bfloat16)],
    )(h, WA, WB)

    eaT = edge_attr.T
    be1r = be1.reshape(1, HID)
    be2r = be2.reshape(1, HID)
    bc1r = bc1.reshape(1, HID)
    wc2r = Wc2.reshape(1, HID)
    We2b = We2.astype(jnp.bfloat16)
    Wc1b = Wc1.astype(jnp.bfloat16)

    accms, accts = [], []
    off = 0
    for idx, ne in enumerate(SLICES):
        si3 = senders[off:off + ne].reshape(NW, ne // NW // C, C)
        ri3 = receivers[off:off + ne].reshape(NW, ne // NW // C, C)
        g, cd, rad = _gathers[idx](hA, hB, y16, si3, ri3)
        mij, w = _edge_mlp(g, eaT[:, off:off + ne], rad, Wea17, be1r,
                           We2b, be2r, Wc1b, bc1r, wc2r)
        accm, acct = _scatters[idx](ri3, mij, w, cd)
        accms.append(accm)
        accts.append(acct)
        off += ne

    def _core_spec(width, core):
        return pl.BlockSpec((1, BN, width),
                            lambda i, core=core: (core, i, 0))

    acc_specs = [_core_spec(HID, c) for _ in SLICES for c in range(NC)]
    acc_specs += [_core_spec(16, c) for _ in SLICES for c in range(NC)]
    acc_args = [a for a in accms for _ in range(NC)]
    acc_args += [a for a in accts for _ in range(NC)]

    h_out, y4_out = pl.pallas_call(
        _node_body,
        grid=(N // BN,),
        in_specs=[pl.BlockSpec((BN, D), lambda i: (i, 0)),
                  pl.BlockSpec((BN, 4), lambda i: (i, 0))]
        + acc_specs
        + [_full((D, HID)), _full((HID, HID)), _full((1, HID)),
           _full((HID, HID)), _full((1, HID))],
        out_specs=[pl.BlockSpec((BN, HID), lambda i: (i, 0)),
                   pl.BlockSpec((BN, 4), lambda i: (i, 0))],
        out_shape=[jax.ShapeDtypeStruct((N, HID), jnp.float32),
                   jax.ShapeDtypeStruct((N, 4), jnp.float32)],
    )(h, y4, *acc_args,
      Wn1[:D], Wn1[D:], bn1.reshape(1, HID), Wn2, bn2.reshape(1, HID))

    return (h_out, y4_out[:, :3], edge_attr)


# speedup vs baseline: 1.5260x; 1.5260x over previous
import functools
import math

import jax
import jax.numpy as jnp
from jax import lax
from jax.experimental import pallas as pl
from jax.experimental.pallas import tpu as pltpu
from jax.experimental.pallas import tpu_sc as plsc

N = 10000
E = 320000
D = 128
DE = 16
HID = 128

NC = 2
NS = 16
NW = NC * NS
C = 80
SLICES = (166400, 153600)
BE = 3200
NACC = 10240
NPT = NACC // NS

_mesh = plsc.VectorSubcoreMesh(core_axis_name="c", subcore_axis_name="s")
_sc_params = pltpu.CompilerParams(needs_layout_passes=False,
                                  use_tc_tiling_on_sc=False)


def _make_gather(ne):
    ept = ne // NW
    nchunk = ept // C

    @functools.partial(
        pl.kernel,
        out_type=(
            jax.ShapeDtypeStruct((ne, D), jnp.float32),
            jax.ShapeDtypeStruct((ne * 4,), jnp.float32),
            jax.ShapeDtypeStruct((ne,), jnp.float32),
        ),
        mesh=_mesh,
        scratch_types=[
            pltpu.VMEM((nchunk, C), jnp.int32),
            pltpu.VMEM((nchunk, C), jnp.int32),
            [pltpu.VMEM((C, D), jnp.float32)] * 2,
            [pltpu.VMEM((C, D), jnp.float32)] * 2,
            [pltpu.VMEM((C, 16), jnp.float32)] * 2,
            [pltpu.VMEM((C, 16), jnp.float32)] * 2,
            [pltpu.VMEM((C, D), jnp.float32)] * 2,
            [pltpu.VMEM((C * 4,), jnp.float32)] * 2,
            [pltpu.VMEM((C,), jnp.float32)] * 2,
            [pltpu.SemaphoreType.DMA] * 2,
            [pltpu.SemaphoreType.DMA] * 2,
        ],
        compiler_params=_sc_params,
    )
    def _gather_k(hA, hB, y16, si3, ri3, g_out, cd_out, rad_out,
                  siv, riv, bA, bB, bYs, bYr, gst, cdb, radv, gsem, wsem):
        cid = lax.axis_index("c")
        sid = lax.axis_index("s")
        wid = sid * NC + cid
        base = wid * ept
        pltpu.sync_copy(si3.at[wid], siv)
        pltpu.sync_copy(ri3.at[wid], riv)

        def issue(k, s):
            pltpu.async_copy(hA.at[siv.at[k]], bA[s], gsem[s])
            pltpu.async_copy(hB.at[riv.at[k]], bB[s], gsem[s])
            pltpu.async_copy(y16.at[siv.at[k]], bYs[s], gsem[s])
            pltpu.async_copy(y16.at[riv.at[k]], bYr[s], gsem[s])

        def step(k, s):
            pltpu.make_async_copy(hA.at[pl.ds(0, C)], bA[s], gsem[s]).wait()
            pltpu.make_async_copy(hB.at[pl.ds(0, C)], bB[s], gsem[s]).wait()
            pltpu.make_async_copy(y16.at[pl.ds(0, C)], bYs[s], gsem[s]).wait()
            pltpu.make_async_copy(y16.at[pl.ds(0, C)], bYr[s], gsem[s]).wait()

            @pl.when(k >= 2)
            def _():
                pltpu.make_async_copy(gst[s], g_out.at[pl.ds(0, C)],
                                      wsem[s]).wait()
                pltpu.make_async_copy(cdb[s], cd_out.at[pl.ds(0, C * 4)],
                                      wsem[s]).wait()
                pltpu.make_async_copy(radv[s], rad_out.at[pl.ds(0, C)],
                                      wsem[s]).wait()

            def cgrp(t, carry2):
                rows = lax.iota(jnp.int32, 16) + t * 16
                rad = jnp.zeros((16,), jnp.float32)
                for comp in range(3):
                    cvec = jnp.full((16,), comp, jnp.int32)
                    ys = plsc.load_gather(bYs[s], [rows, cvec])
                    yr = plsc.load_gather(bYr[s], [rows, cvec])
                    dd = yr - ys
                    plsc.store_scatter(cdb[s], [rows * 4 + comp], dd)
                    rad = rad + dd * dd
                plsc.store_scatter(cdb[s], [rows * 4 + 3], rad)
                radv[s][pl.ds(t * 16, 16)] = rad
                return carry2
            lax.fori_loop(0, C // 16, cgrp, 0)

            def addrow(i, carry2):
                for j in range(D // 16):
                    gst[s][i, pl.ds(j * 16, 16)] = (
                        bA[s][i, pl.ds(j * 16, 16)]
                        + bB[s][i, pl.ds(j * 16, 16)])
                return carry2
            lax.fori_loop(0, C, addrow, 0)

            off = base + k * C
            pltpu.async_copy(gst[s], g_out.at[pl.ds(off, C)], wsem[s])
            pltpu.async_copy(cdb[s], cd_out.at[pl.ds(off * 4, C * 4)],
                             wsem[s])
            pltpu.async_copy(radv[s], rad_out.at[pl.ds(off, C)], wsem[s])

            @pl.when(k + 2 < nchunk)
            def _():
                issue(k + 2, s)

        issue(0, 0)
        issue(1, 1)

        def pair(i, carry):
            step(2 * i, 0)

            @pl.when(2 * i + 1 < nchunk)
            def _():
                step(2 * i + 1, 1)
            return carry
        lax.fori_loop(0, (nchunk + 1) // 2, pair, 0)
        for s in range(2):
            pltpu.make_async_copy(gst[s], g_out.at[pl.ds(0, C)],
                                  wsem[s]).wait()
            pltpu.make_async_copy(cdb[s], cd_out.at[pl.ds(0, C * 4)],
                                  wsem[s]).wait()
            pltpu.make_async_copy(radv[s], rad_out.at[pl.ds(0, C)],
                                  wsem[s]).wait()

    return _gather_k


_gathers = tuple(_make_gather(ne) for ne in SLICES)


def _make_scatter(ne):
    ept = ne // NW
    nchunk = ept // C

    @functools.partial(
        pl.kernel,
        out_type=(
            jax.ShapeDtypeStruct((NC, NACC, HID), jnp.float32),
            jax.ShapeDtypeStruct((NC, NACC, 16), jnp.float32),
        ),
        mesh=_mesh,
        scratch_types=[
            pltpu.VMEM((nchunk, C), jnp.int32),
            [pltpu.VMEM((C, HID), jnp.float32)] * 2,
            [pltpu.VMEM((C, 16), jnp.float32)] * 2,
            [pltpu.VMEM((C,), jnp.float32)] * 2,
            [pltpu.VMEM((C * 4,), jnp.float32)] * 2,
            [pltpu.SemaphoreType.DMA] * 2,
            pltpu.VMEM_SHARED((NACC, HID), jnp.float32),
            pltpu.VMEM_SHARED((NACC, 16), jnp.float32),
        ],
        compiler_params=_sc_params,
    )
    def _scatter_k(ri3, mij, w, cd, accm_out, acct_out,
                   riv, mb, tb, wb, cb, lsem, accm_sh, acct_sh):
        cid = lax.axis_index("c")
        sid = lax.axis_index("s")
        wid = sid * NC + cid
        base = wid * ept
        pltpu.sync_copy(ri3.at[wid], riv)

        def zrow(i, carry):
            for j in range(HID // 16):
                mb[0][i, pl.ds(j * 16, 16)] = jnp.zeros((16,), jnp.float32)
            tb[0][i, pl.ds(0, 16)] = jnp.zeros((16,), jnp.float32)
            tb[1][i, pl.ds(0, 16)] = jnp.zeros((16,), jnp.float32)
            return carry
        lax.fori_loop(0, C, zrow, 0)

        for q in range(NPT // C):
            pltpu.sync_copy(mb[0], accm_sh.at[pl.ds(sid * NPT + q * C, C)])
            pltpu.sync_copy(tb[0], acct_sh.at[pl.ds(sid * NPT + q * C, C)])
        plsc.subcore_barrier()

        def issue(k, s):
            off = base + k * C
            pltpu.async_copy(mij.at[pl.ds(off, C)], mb[s], lsem[s])
            pltpu.async_copy(w.at[pl.ds(off, C)], wb[s], lsem[s])
            pltpu.async_copy(cd.at[pl.ds(off * 4, C * 4)], cb[s], lsem[s])

        def step(k, s):
            pltpu.make_async_copy(mij.at[pl.ds(0, C)], mb[s], lsem[s]).wait()
            pltpu.make_async_copy(w.at[pl.ds(0, C)], wb[s], lsem[s]).wait()
            pltpu.make_async_copy(cd.at[pl.ds(0, C * 4)], cb[s],
                                  lsem[s]).wait()

            def tgrp(t, carry2):
                rows = lax.iota(jnp.int32, 16) + t * 16
                wv = wb[s][pl.ds(t * 16, 16)]
                for comp in range(3):
                    dd = plsc.load_gather(cb[s], [rows * 4 + comp])
                    plsc.store_scatter(
                        tb[s], [rows, jnp.full((16,), comp, jnp.int32)],
                        dd * wv)
                plsc.store_scatter(
                    tb[s], [rows, jnp.full((16,), 3, jnp.int32)],
                    jnp.full((16,), 1.0, jnp.float32))
                return carry2
            lax.fori_loop(0, C // 16, tgrp, 0)

            pltpu.sync_copy(mb[s], accm_sh.at[riv.at[k]], add=True)
            pltpu.sync_copy(tb[s], acct_sh.at[riv.at[k]], add=True)

            @pl.when(k + 2 < nchunk)
            def _():
                issue(k + 2, s)

        issue(0, 0)
        issue(1, 1)

        def pair(i, carry):
            step(2 * i, 0)

            @pl.when(2 * i + 1 < nchunk)
            def _():
                step(2 * i + 1, 1)
            return carry
        lax.fori_loop(0, (nchunk + 1) // 2, pair, 0)
        plsc.subcore_barrier()

        for q in range(NPT // C):
            rows = sid * NPT + q * C
            pltpu.sync_copy(accm_sh.at[pl.ds(rows, C)], mb[0])
            pltpu.sync_copy(mb[0], accm_out.at[cid, pl.ds(rows, C)])
            pltpu.sync_copy(acct_sh.at[pl.ds(rows, C)], tb[0])
            pltpu.sync_copy(tb[0], acct_out.at[cid, pl.ds(rows, C)])

    return _scatter_k


_scatters = tuple(_make_scatter(ne) for ne in SLICES)


def _pre_body(h_ref, wa_ref, wb_ref, ha_ref, hb_ref):
    h = h_ref[...]
    ha_ref[...] = jnp.dot(h, wa_ref[...], preferred_element_type=jnp.float32)
    hb_ref[...] = jnp.dot(h, wb_ref[...], preferred_element_type=jnp.float32)


def _edge_body(g_ref, eat_ref, rad_ref, wea_ref, be1_ref,
               we2_ref, be2_ref, wc1_ref, bc1_ref, wc2_ref,
               mij_ref, w_ref):
    g = g_ref[...]
    be = g.shape[0]
    i = pl.program_id(0)
    rad_row = rad_ref[pl.ds(i * be, be)].reshape(1, be)
    ea17 = jnp.concatenate([eat_ref[...], rad_row], axis=0)
    pre1 = (g
            + jax.lax.dot_general(ea17, wea_ref[...],
                                  (((0,), (0,)), ((), ())),
                                  preferred_element_type=jnp.float32)
            + be1_ref[...])
    m1 = jax.nn.silu(pre1)
    mij = jax.nn.silu(
        jnp.dot(m1.astype(jnp.bfloat16), we2_ref[...],
                preferred_element_type=jnp.float32)
        + be2_ref[...])
    cvec = jax.nn.silu(
        jnp.dot(mij.astype(jnp.bfloat16), wc1_ref[...],
                preferred_element_type=jnp.float32)
        + bc1_ref[...])
    w_row = jax.lax.dot_general(wc2_ref[...], cvec,
                                (((1,), (1,)), ((), ())),
                                preferred_element_type=jnp.float32)
    mij_ref[...] = mij
    w_ref[pl.ds(i * be, be)] = w_row.reshape(be)


def _node_body(h_ref, y4_ref, *rest):
    k = 2 * len(SLICES)
    ams = rest[:k]
    ats = rest[k:2 * k]
    wn1t_ref, wn1b_ref, bn1_ref, wn2_ref, bn2_ref, hout_ref, yout_ref = \
        rest[2 * k:]
    h = h_ref[...]
    mi = ams[0][0]
    for r in ams[1:]:
        mi = mi + r[0]
    mi = mi * (1.0 / math.sqrt(648.0))
    u = jax.nn.silu(
        jnp.dot(h, wn1t_ref[...], preferred_element_type=jnp.float32)
        + jnp.dot(mi, wn1b_ref[...], preferred_element_type=jnp.float32)
        + bn1_ref[...])
    hout_ref[...] = (h + jnp.dot(u, wn2_ref[...],
                                 preferred_element_type=jnp.float32)
                     + bn2_ref[...])
    t = ats[0][0]
    for r in ats[1:]:
        t = t + r[0]
    cnt = jnp.maximum(t[:, 3:4], 1.0)
    yout_ref[...] = y4_ref[...] + t[:, :4] / cnt


def _full(shape):
    return pl.BlockSpec(shape, lambda i: (0,) * len(shape))


def _edge_mlp(g, eaT, rad, Wea17, be1, We2, be2, Wc1, bc1, Wc2):
    ne = g.shape[0]
    return pl.pallas_call(
        _edge_body,
        grid=(ne // BE,),
        in_specs=[pl.BlockSpec((BE, HID), lambda i: (i, 0)),
                  pl.BlockSpec((DE, BE), lambda i: (0, i)),
                  pl.BlockSpec((ne,), lambda i: (0,)),
                  _full((DE + 1, HID)), _full((1, HID)),
                  pl.BlockSpec((HID, HID), lambda i: (0, 0)),
                  _full((1, HID)),
                  pl.BlockSpec((HID, HID), lambda i: (0, 0)),
                  _full((1, HID)), _full((1, HID))],
        out_specs=[pl.BlockSpec((BE, HID), lambda i: (i, 0)),
                   pl.BlockSpec((ne,), lambda i: (0,))],
        out_shape=[jax.ShapeDtypeStruct((ne, HID), jnp.float32),
                   jax.ShapeDtypeStruct((ne,), jnp.float32)],
    )(g, eaT, rad, Wea17, be1, We2, be2, Wc1, bc1, Wc2)


def kernel(h, edge_index, y, edge_attr, We1, be1, We2, be2,
           Wc1, bc1, Wc2, Wn1, bn1, Wn2, bn2):
    receivers = edge_index[0].astype(jnp.int32)
    senders = edge_index[1].astype(jnp.int32)
    y4 = jnp.pad(y, ((0, 0), (0, 1)))
    y16 = jnp.pad(y, ((0, 0), (0, 13)))

    WA = We1[:D]
    WB = We1[D:2 * D]
    Wea17 = jnp.concatenate([We1[2 * D + 1:], We1[2 * D:2 * D + 1]], axis=0)

    BN = 2000
    hA, hB = pl.pallas_call(
        _pre_body,
        grid=(N // BN,),
        in_specs=[pl.BlockSpec((BN, D), lambda i: (i, 0)),
                  _full((D, HID)), _full((D, HID))],
        out_specs=[pl.BlockSpec((BN, HID), lambda i: (i, 0)),
                   pl.BlockSpec((BN, HID), lambda i: (i, 0))],
        out_shape=[jax.ShapeDtypeStruct((N, HID), jnp.float32),
                   jax.ShapeDtypeStruct((N, HID), jnp.float32)],
    )(h, WA, WB)

    eaT = edge_attr.T
    be1r = be1.reshape(1, HID)
    be2r = be2.reshape(1, HID)
    bc1r = bc1.reshape(1, HID)
    wc2r = Wc2.reshape(1, HID)
    We2b = We2.astype(jnp.bfloat16)
    Wc1b = Wc1.astype(jnp.bfloat16)

    accms, accts = [], []
    off = 0
    for idx, ne in enumerate(SLICES):
        si3 = senders[off:off + ne].reshape(NW, ne // NW // C, C)
        ri3 = receivers[off:off + ne].reshape(NW, ne // NW // C, C)
        g, cd, rad = _gathers[idx](hA, hB, y16, si3, ri3)
        mij, w = _edge_mlp(g, eaT[:, off:off + ne], rad, Wea17, be1r,
                           We2b, be2r, Wc1b, bc1r, wc2r)
        accm, acct = _scatters[idx](ri3, mij, w, cd)
        accms.append(accm)
        accts.append(acct)
        off += ne

    def _core_spec(width, core):
        return pl.BlockSpec((1, BN, width),
                            lambda i, core=core: (core, i, 0))

    acc_specs = [_core_spec(HID, c) for _ in SLICES for c in range(NC)]
    acc_specs += [_core_spec(16, c) for _ in SLICES for c in range(NC)]
    acc_args = [a for a in accms for _ in range(NC)]
    acc_args += [a for a in accts for _ in range(NC)]

    h_out, y4_out = pl.pallas_call(
        _node_body,
        grid=(N // BN,),
        in_specs=[pl.BlockSpec((BN, D), lambda i: (i, 0)),
                  pl.BlockSpec((BN, 4), lambda i: (i, 0))]
        + acc_specs
        + [_full((D, HID)), _full((HID, HID)), _full((1, HID)),
           _full((HID, HID)), _full((1, HID))],
        out_specs=[pl.BlockSpec((BN, HID), lambda i: (i, 0)),
                   pl.BlockSpec((BN, 4), lambda i: (i, 0))],
        out_shape=[jax.ShapeDtypeStruct((N, HID), jnp.float32),
                   jax.ShapeDtypeStruct((N, 4), jnp.float32)],
    )(h, y4, *acc_args,
      Wn1[:D], Wn1[D:], bn1.reshape(1, HID), Wn2, bn2.reshape(1, HID))

    return (h_out, y4_out[:, :3], edge_attr)
